# baseline (speedup 1.0000x reference)
import functools

import jax
import jax.numpy as jnp
from jax import lax
from jax.experimental import pallas as pl
from jax.experimental.pallas import tpu as pltpu
from jax.experimental.pallas import tpu_sc as plsc

B, C, H, W = 128, 768, 14, 14
E = 64
TOP_K = 8
HW = H * W
HW_BLK = 14
NEG = -3.0e38

N_CORES = 2
N_SUBCORES = 16
N_WORKERS = N_CORES * N_SUBCORES
ROWS_PER_WORKER = B // N_WORKERS
N_VREG = E // 16


def _pool_matmul_body(x_ref, w_ref, out_ref, acc_ref):
    i = pl.program_id(0)

    @pl.when(i == 0)
    def _init():
        acc_ref[...] = jnp.zeros_like(acc_ref)

    acc_ref[...] += jnp.sum(x_ref[...], axis=0)

    @pl.when(i == pl.num_programs(0) - 1)
    def _finish():
        pooled = acc_ref[...] * jnp.float32(1.0 / HW)
        out_ref[...] = jnp.dot(pooled, w_ref[...],
                               preferred_element_type=jnp.float32)


def _sc_gating_body(logits_hbm, out_hbm, lrows, orows):
    wid = lax.axis_index("s") * N_CORES + lax.axis_index("c")
    base = wid * ROWS_PER_WORKER
    pltpu.sync_copy(logits_hbm.at[pl.ds(base, ROWS_PER_WORKER)], lrows)
    lanes = lax.iota(jnp.int32, 16)
    for r in range(ROWS_PER_WORKER):
        v = [lrows[r, pl.ds(j * 16, 16)] for j in range(N_VREG)]
        avail = list(v)
        sel = [jnp.zeros((16,), jnp.bool_) for _ in range(N_VREG)]
        row_max = jnp.float32(NEG)
        for k in range(TOP_K):
            m = avail[0]
            for j in range(1, N_VREG):
                m = jnp.maximum(m, avail[j])
            mmax = jnp.max(m)
            if k == 0:
                row_max = mmax
            pos = jnp.int32(E)
            for j in range(N_VREG):
                ffs = plsc.all_reduce_ffs(avail[j] == mmax)
                cand = jnp.where(ffs < 16, jnp.int32(j * 16) + ffs,
                                 jnp.int32(E))
                pos = jnp.minimum(pos, cand)
            for j in range(N_VREG):
                hit = lanes == (pos - jnp.int32(j * 16))
                sel[j] = sel[j] | hit
                avail[j] = jnp.where(hit, jnp.float32(NEG), avail[j])
        ex = [jnp.where(sel[j], jnp.exp(v[j] - row_max), jnp.float32(0.0))
              for j in range(N_VREG)]
        s_vec = ex[0]
        for j in range(1, N_VREG):
            s_vec = s_vec + ex[j]
        denom = jnp.broadcast_to(jnp.sum(s_vec), (16,))
        for j in range(N_VREG):
            orows[r, pl.ds(j * 16, 16)] = ex[j] / denom
    pltpu.sync_copy(orows, out_hbm.at[pl.ds(base, ROWS_PER_WORKER)])


_sc_gating = functools.partial(
    pl.kernel,
    out_type=jax.ShapeDtypeStruct((B, E), jnp.float32),
    mesh=plsc.VectorSubcoreMesh(core_axis_name="c", subcore_axis_name="s"),
    compiler_params=pltpu.CompilerParams(needs_layout_passes=False),
    scratch_types=[
        pltpu.VMEM((ROWS_PER_WORKER, E), jnp.float32),
        pltpu.VMEM((ROWS_PER_WORKER, E), jnp.float32),
    ],
)(_sc_gating_body)


@jax.jit
def kernel(x, W_gate):
    xs = jnp.transpose(x, (2, 3, 0, 1)).reshape(HW, B, C)
    logits = pl.pallas_call(
        _pool_matmul_body,
        grid=(HW // HW_BLK,),
        in_specs=[
            pl.BlockSpec((HW_BLK, B, C), lambda i: (i, 0, 0)),
            pl.BlockSpec((C, E), lambda i: (0, 0)),
        ],
        out_specs=pl.BlockSpec((B, E), lambda i: (0, 0)),
        out_shape=jax.ShapeDtypeStruct((B, E), jnp.float32),
        scratch_shapes=[pltpu.VMEM((B, C), jnp.float32)],
    )(xs, W_gate)
    return _sc_gating(logits)
